# Initial kernel scaffold; baseline (speedup 1.0000x reference)
import jax
import jax.numpy as jnp
from jax.experimental import pallas as pl


def kernel(x, pos_embedding):
    raise NotImplementedError("write your pallas kernel here")


# speedup vs baseline: 1.4415x; 1.4415x over previous
import jax
import jax.numpy as jnp
from jax.experimental import pallas as pl
from jax.experimental.pallas import tpu as pltpu


def _add_body(x_ref, pos_ref, out_ref):
    out_ref[...] = x_ref[...] + pos_ref[...]


def kernel(x, pos_embedding):
    batch, seq_len, emb = x.shape
    TS = 512
    ns = seq_len // TS
    grid = (ns, batch)
    return pl.pallas_call(
        _add_body,
        grid=grid,
        in_specs=[
            pl.BlockSpec((1, TS, emb), lambda s, b: (b, s, 0)),
            pl.BlockSpec((TS, emb), lambda s, b: (s, 0)),
        ],
        out_specs=pl.BlockSpec((1, TS, emb), lambda s, b: (b, s, 0)),
        out_shape=jax.ShapeDtypeStruct((batch, seq_len, emb), x.dtype),
    )(x, pos_embedding)
